# baseline (speedup 1.0000x reference)
import functools

import jax
import jax.numpy as jnp
from jax import lax
from jax.experimental import pallas as pl
from jax.experimental.pallas import tpu as pltpu
from jax.experimental.pallas import tpu_sc as plsc

_NC = 2
_NS = 16
_NW = _NC * _NS
_CH = 65536
_NB = 3
_BR = 4096


def _make_sc_copy(n):
    per_w = n // _NW
    chunks_per_tensor = per_w // _CH
    mesh = plsc.VectorSubcoreMesh(
        core_axis_name="c", subcore_axis_name="s",
        num_cores=_NC, num_subcores=_NS)

    @functools.partial(
        pl.kernel,
        out_type=jax.ShapeDtypeStruct((n,), jnp.bfloat16),
        mesh=mesh,
        scratch_types=(
            [pltpu.VMEM((_CH,), jnp.bfloat16)] * _NB
            + [pltpu.SemaphoreType.DMA] * (2 * _NB)
        ),
    )
    def sc_copy(v_hbm, vo_hbm, b0, b1, b2, si0, si1, si2, so0, so1, so2):
        bufs = (b0, b1, b2)
        sin = (si0, si1, si2)
        sout = (so0, so1, so2)
        wid = lax.axis_index("s") * _NC + lax.axis_index("c")
        base = wid * per_w

        ins, outs = [], []
        for j in range(chunks_per_tensor):
            b = j % _NB
            sl = pl.ds(base + j * _CH, _CH)
            ins.append(pltpu.make_async_copy(v_hbm.at[sl], bufs[b], sin[b]))
            outs.append(pltpu.make_async_copy(bufs[b], vo_hbm.at[sl], sout[b]))

        nj = chunks_per_tensor
        for j in range(min(_NB, nj)):
            ins[j].start()
        for j in range(nj):
            ins[j].wait()
            outs[j].start()
            nxt = j + _NB
            if nxt < nj:
                outs[j].wait()
                ins[nxt].start()
        for j in range(max(0, nj - _NB), nj):
            outs[j].wait()

    return sc_copy


def _tc_body(k_ref, ko_ref):
    ko_ref[...] = k_ref[...]


def _tc_copy(k):
    R, D = k.shape
    spec = pl.BlockSpec((_BR, D), lambda i: (i, 0))
    return pl.pallas_call(
        _tc_body,
        grid=(R // _BR,),
        in_specs=[spec],
        out_specs=spec,
        out_shape=jax.ShapeDtypeStruct((R, D), jnp.bfloat16),
        compiler_params=pltpu.CompilerParams(
            dimension_semantics=("arbitrary",)),
    )(k)


def kernel(key_states, value_states, k_cache, v_cache, layer_idx):
    B, H, S, D = key_states.shape
    n = B * H * S * D
    k = lax.bitcast_convert_type(key_states, jnp.bfloat16).reshape(n // D, D)
    v = lax.bitcast_convert_type(value_states, jnp.bfloat16).reshape(n)
    ko = _tc_copy(k)
    vo = _make_sc_copy(n)(v)
    ko = lax.bitcast_convert_type(ko.reshape(B, H, S, D), jnp.float16)
    vo = lax.bitcast_convert_type(vo.reshape(B, H, S, D), jnp.float16)
    return ko, vo


# speedup vs baseline: 1.0924x; 1.0924x over previous
import functools

import jax
import jax.numpy as jnp
from jax import lax
from jax.experimental import pallas as pl
from jax.experimental.pallas import tpu as pltpu
from jax.experimental.pallas import tpu_sc as plsc

_NC = 2
_NS = 16
_NW = _NC * _NS
_CHR = 512
_NB = 3


def _make_sc_copy(rows, d):
    rows_per_w = rows // _NW
    nj_per_tensor = rows_per_w // _CHR
    mesh = plsc.VectorSubcoreMesh(
        core_axis_name="c", subcore_axis_name="s",
        num_cores=_NC, num_subcores=_NS)

    @functools.partial(
        pl.kernel,
        out_type=[jax.ShapeDtypeStruct((rows, d), jnp.bfloat16)] * 2,
        mesh=mesh,
        scratch_types=(
            [pltpu.VMEM((_CHR, d), jnp.bfloat16)] * _NB
            + [pltpu.SemaphoreType.DMA] * (2 * _NB)
        ),
    )
    def sc_copy(k_hbm, v_hbm, ko_hbm, vo_hbm,
                b0, b1, b2, si0, si1, si2, so0, so1, so2):
        bufs = (b0, b1, b2)
        sin = (si0, si1, si2)
        sout = (so0, so1, so2)
        wid = lax.axis_index("s") * _NC + lax.axis_index("c")
        base = wid * rows_per_w

        jobs = []
        for src, dst in ((k_hbm, ko_hbm), (v_hbm, vo_hbm)):
            for c in range(nj_per_tensor):
                jobs.append((src, dst, c * _CHR))
        ins, outs = [], []
        for j, (src, dst, off) in enumerate(jobs):
            b = j % _NB
            sl = pl.ds(base + off, _CHR)
            ins.append(pltpu.make_async_copy(src.at[sl], bufs[b], sin[b]))
            outs.append(pltpu.make_async_copy(bufs[b], dst.at[sl], sout[b]))

        nj = len(jobs)
        for j in range(min(_NB, nj)):
            ins[j].start()
        for j in range(nj):
            ins[j].wait()
            outs[j].start()
            nxt = j + _NB
            if nxt < nj:
                outs[j].wait()
                ins[nxt].start()
        for j in range(max(0, nj - _NB), nj):
            outs[j].wait()

    return sc_copy


def kernel(key_states, value_states, k_cache, v_cache, layer_idx):
    B, H, S, D = key_states.shape
    rows = B * H * S
    k = lax.bitcast_convert_type(key_states, jnp.bfloat16).reshape(rows, D)
    v = lax.bitcast_convert_type(value_states, jnp.bfloat16).reshape(rows, D)
    ko, vo = _make_sc_copy(rows, D)(k, v)
    ko = lax.bitcast_convert_type(ko.reshape(B, H, S, D), jnp.float16)
    vo = lax.bitcast_convert_type(vo.reshape(B, H, S, D), jnp.float16)
    return ko, vo


# speedup vs baseline: 2.3263x; 2.1295x over previous
import functools

import jax
import jax.numpy as jnp
from jax import lax
from jax.experimental import pallas as pl
from jax.experimental.pallas import tpu as pltpu
from jax.experimental.pallas import tpu_sc as plsc

_NC = 2
_NS = 16
_NW = _NC * _NS
_CHR = 512
_NB = 3


def _make_sc_copy(rows, d):
    rows_per_w = rows // _NW
    nj_per_tensor = rows_per_w // _CHR
    mesh = plsc.VectorSubcoreMesh(
        core_axis_name="c", subcore_axis_name="s",
        num_cores=_NC, num_subcores=_NS)

    @functools.partial(
        pl.kernel,
        out_type=[jax.ShapeDtypeStruct((rows, d), jnp.float16)] * 2,
        mesh=mesh,
        scratch_types=(
            [pltpu.VMEM((_CHR, d), jnp.float16)] * _NB
            + [pltpu.SemaphoreType.DMA] * (2 * _NB)
        ),
    )
    def sc_copy(k_hbm, v_hbm, ko_hbm, vo_hbm,
                b0, b1, b2, si0, si1, si2, so0, so1, so2):
        bufs = (b0, b1, b2)
        sin = (si0, si1, si2)
        sout = (so0, so1, so2)
        wid = lax.axis_index("s") * _NC + lax.axis_index("c")
        base = wid * rows_per_w

        jobs = []
        for src, dst in ((k_hbm, ko_hbm), (v_hbm, vo_hbm)):
            for c in range(nj_per_tensor):
                jobs.append((src, dst, c * _CHR))
        ins, outs = [], []
        for j, (src, dst, off) in enumerate(jobs):
            b = j % _NB
            sl = pl.ds(base + off, _CHR)
            ins.append(pltpu.make_async_copy(src.at[sl], bufs[b], sin[b]))
            outs.append(pltpu.make_async_copy(bufs[b], dst.at[sl], sout[b]))

        nj = len(jobs)
        for j in range(min(_NB, nj)):
            ins[j].start()
        for j in range(nj):
            ins[j].wait()
            outs[j].start()
            nxt = j + _NB
            if nxt < nj:
                outs[j].wait()
                ins[nxt].start()
        for j in range(max(0, nj - _NB), nj):
            outs[j].wait()

    return sc_copy


def kernel(key_states, value_states, k_cache, v_cache, layer_idx):
    B, H, S, D = key_states.shape
    rows = B * H * S
    k = key_states.reshape(rows, D)
    v = value_states.reshape(rows, D)
    ko, vo = _make_sc_copy(rows, D)(k, v)
    return ko.reshape(B, H, S, D), vo.reshape(B, H, S, D)
